# baseline (speedup 1.0000x reference)
import functools

import jax
import jax.numpy as jnp
from jax import lax
from jax.experimental import pallas as pl
from jax.experimental.pallas import tpu as pltpu
from jax.experimental.pallas import tpu_sc as plsc

_BATCH = 4096
_SEQ = 200
_D = 64
_NC, _NS = 2, 16
_NW = _NC * _NS
_RPW = _BATCH // _NW
_SPLITS = ((0, 104), (104, 96))
_RB = 4
_NCH = _RPW // _RB


def _fire_gathers(table, idx_v, r0, rows_b, gsem):
    copies = []
    for rb in range(_RB):
        for off, ln in _SPLITS:
            copies.append(pltpu.async_copy(
                table.at[idx_v.at[r0 + rb, pl.ds(off, ln)]],
                rows_b.at[rb, pl.ds(off, ln)], gsem))
    return copies


def _sc_body(p_idx, h_idx, table, p_out, h_out,
             idx_v, rows0, rows1, gsem0, gsem1, osem0, osem1):
    wid = lax.axis_index("s") * _NC + lax.axis_index("c")
    rows = (rows0, rows1)
    gsems = (gsem0, gsem1)
    osems = (osem0, osem1)
    gbase = wid * _RPW

    for src, dst in ((p_idx, p_out), (h_idx, h_out)):
        pltpu.sync_copy(src.at[pl.ds(gbase, _RPW)], idx_v)

        for b in range(2):
            for cp in _fire_gathers(table, idx_v, b * _RB, rows[b], gsems[b]):
                cp.wait()
            pltpu.async_copy(rows[b], dst.at[pl.ds(gbase + b * _RB, _RB)],
                             osems[b])

        @pl.loop(2, _NCH, step=2)
        def _steady(c0):
            for b in range(2):
                c = c0 + b
                pltpu.make_async_copy(
                    rows[b], dst.at[pl.ds(gbase, _RB)], osems[b]).wait()
                for cp in _fire_gathers(table, idx_v, c * _RB, rows[b],
                                        gsems[b]):
                    cp.wait()
                pltpu.async_copy(rows[b], dst.at[pl.ds(gbase + c * _RB, _RB)],
                                 osems[b])

        for b in range(2):
            pltpu.make_async_copy(
                rows[b], dst.at[pl.ds(gbase, _RB)], osems[b]).wait()


@functools.partial(
    pl.kernel,
    out_type=(jax.ShapeDtypeStruct((_BATCH, _SEQ, _D), jnp.float32),
              jax.ShapeDtypeStruct((_BATCH, _SEQ, _D), jnp.float32)),
    mesh=plsc.VectorSubcoreMesh(core_axis_name="c", subcore_axis_name="s"),
    compiler_params=pltpu.CompilerParams(use_tc_tiling_on_sc=False),
    scratch_types=[
        pltpu.VMEM((_RPW, _SEQ), jnp.int32),
        pltpu.VMEM((_RB, _SEQ, _D), jnp.float32),
        pltpu.VMEM((_RB, _SEQ, _D), jnp.float32),
        pltpu.SemaphoreType.DMA,
        pltpu.SemaphoreType.DMA,
        pltpu.SemaphoreType.DMA,
        pltpu.SemaphoreType.DMA,
    ],
)
def _embed_lookup(p_idx, h_idx, table, p_out, h_out,
                  idx_v, rows0, rows1, gsem0, gsem1, osem0, osem1):
    _sc_body(p_idx, h_idx, table, p_out, h_out,
             idx_v, rows0, rows1, gsem0, gsem1, osem0, osem1)


def kernel(premises, hypotheses, labels, table):
    p_emb, h_emb = _embed_lookup(premises, hypotheses, table)
    return (p_emb, h_emb, labels)
